# baseline (speedup 1.0000x reference)
import functools

import jax
import jax.numpy as jnp
from jax import lax
from jax.experimental import pallas as pl
from jax.experimental.pallas import tpu as pltpu
from jax.experimental.pallas import tpu_sc as plsc

BATCH = 1024
SEQ = 200
EMBED_DIM = 32
TOTAL = BATCH * SEQ

NC = 2
NS = 16
NW = NC * NS
CHUNK = 128
ROWS_PER_W = TOTAL // NW
CH_PER_W = ROWS_PER_W // CHUNK

_mesh = plsc.VectorSubcoreMesh(core_axis_name="c", subcore_axis_name="s")


@functools.partial(
    pl.kernel,
    mesh=_mesh,
    compiler_params=pltpu.CompilerParams(use_tc_tiling_on_sc=False),
    out_type=jax.ShapeDtypeStruct((TOTAL, EMBED_DIM), jnp.float32),
    scratch_types=[
        pltpu.VMEM((ROWS_PER_W,), jnp.int32),
        pltpu.VMEM((CHUNK, EMBED_DIM), jnp.float32),
        pltpu.SemaphoreType.DMA,
    ],
)
def _sc_gather(table_hbm, idx_hbm, out_hbm, idx_v, rows_v, sem):
    wid = lax.axis_index("s") * NC + lax.axis_index("c")
    rbase = wid * ROWS_PER_W
    pltpu.sync_copy(idx_hbm.at[pl.ds(rbase, ROWS_PER_W)], idx_v)

    def step(j, carry):
        idx_chunk = idx_v.at[pl.ds(j * CHUNK, CHUNK)]
        pltpu.async_copy(table_hbm.at[idx_chunk], rows_v, sem).wait()
        pltpu.sync_copy(rows_v, out_hbm.at[pl.ds(rbase + j * CHUNK, CHUNK)])
        return carry

    lax.fori_loop(0, CH_PER_W, step, 0)


_TR_BLK = 128


def _tracker_body(tr_ref, ids_ref, out_ref):
    w = pl.program_id(0)
    t = tr_ref[...]
    out_ref[...] = t

    @pl.when(w < BATCH // _TR_BLK)
    def _():
        col = lax.broadcasted_iota(jnp.int32, (_TR_BLK, 256), 1)
        out_ref[:, :256] = jnp.where(col < SEQ, ids_ref[...], t[:, :256])


def _tracker(tr, ids_pad):
    n = tr.shape[0] // _TR_BLK
    return pl.pallas_call(
        _tracker_body,
        grid=(n,),
        in_specs=[
            pl.BlockSpec((_TR_BLK, tr.shape[1]), lambda w: (w, 0)),
            pl.BlockSpec((_TR_BLK, 256), lambda w: (jnp.minimum(w, BATCH // _TR_BLK - 1), 0)),
        ],
        out_specs=pl.BlockSpec((_TR_BLK, tr.shape[1]), lambda w: (w, 0)),
        out_shape=jax.ShapeDtypeStruct(tr.shape, jnp.int32),
    )(tr, ids_pad)


def kernel(inp_ids, table, idx_tracker):
    ids32 = inp_ids.astype(jnp.int32)
    idx_flat = ids32.reshape(TOTAL)
    out = _sc_gather(table, idx_flat).reshape(BATCH, SEQ, EMBED_DIM)
    ids_pad = jnp.pad(ids32, ((0, 0), (0, 256 - SEQ)))
    tracker = _tracker(idx_tracker.astype(jnp.int32), ids_pad).astype(idx_tracker.dtype)
    return out, tracker
